# baseline (speedup 1.0000x reference)
import functools

import jax
import jax.numpy as jnp
from jax import lax
from jax.experimental import pallas as pl
from jax.experimental.pallas import tpu as pltpu
from jax.experimental.pallas import tpu_sc as plsc

_NW = 32
_CH = 128


def _sc_gather_rows(weight2, sample_ids, labels):
    V2, D2 = weight2.shape
    S = sample_ids.shape[0]
    B = labels.shape[0]
    s_per = S // _NW
    b_per = B // _NW
    mesh = plsc.VectorSubcoreMesh(core_axis_name="c", subcore_axis_name="s")

    @functools.partial(
        pl.kernel,
        mesh=mesh,
        out_type=(
            jax.ShapeDtypeStruct((S, D2), jnp.float32),
            jax.ShapeDtypeStruct((B, D2), jnp.float32),
        ),
        scratch_types=(
            pltpu.VMEM((s_per,), jnp.int32),
            pltpu.VMEM((b_per,), jnp.int32),
            pltpu.VMEM((s_per,), jnp.int32),
            pltpu.VMEM((b_per,), jnp.int32),
            pltpu.VMEM((s_per, D2), jnp.float32),
            pltpu.VMEM((b_per, D2), jnp.float32),
            pltpu.SemaphoreType.DMA,
        ),
    )
    def gk(w_hbm, sid_hbm, lab_hbm, sw_hbm, tw_hbm,
           sidx_v, lidx_v, shalf_v, lhalf_v, srows_v, lrows_v, sem):
        wid = lax.axis_index("s") * 2 + lax.axis_index("c")
        sbase = wid * s_per
        lbase = wid * b_per
        pltpu.sync_copy(sid_hbm.at[pl.ds(sbase, s_per)], sidx_v)
        pltpu.sync_copy(lab_hbm.at[pl.ds(lbase, b_per)], lidx_v)
        for k in range(s_per // 16):
            sl = pl.ds(k * 16, 16)
            shalf_v[sl] = lax.shift_right_logical(sidx_v[sl], 1)
        for k in range(b_per // 16):
            sl = pl.ds(k * 16, 16)
            lhalf_v[sl] = lax.shift_right_logical(lidx_v[sl], 1)
        cps = []
        for k in range(s_per // _CH):
            sl = pl.ds(k * _CH, _CH)
            cps.append(pltpu.async_copy(
                w_hbm.at[shalf_v.at[sl]], srows_v.at[sl], sem))
        for k in range(b_per // _CH):
            sl = pl.ds(k * _CH, _CH)
            cps.append(pltpu.async_copy(
                w_hbm.at[lhalf_v.at[sl]], lrows_v.at[sl], sem))
        for cp in cps:
            cp.wait()
        pltpu.sync_copy(srows_v, sw_hbm.at[pl.ds(sbase, s_per)])
        pltpu.sync_copy(lrows_v, tw_hbm.at[pl.ds(lbase, b_per)])

    return gk(weight2, sample_ids, labels)


def _sc_gather_bias(bias, sample_ids, labels):
    S = sample_ids.shape[0]
    B = labels.shape[0]
    s_per = S // _NW
    b_per = B // _NW
    mesh = plsc.VectorSubcoreMesh(core_axis_name="c", subcore_axis_name="s")

    @functools.partial(
        pl.kernel,
        mesh=mesh,
        compiler_params=pltpu.CompilerParams(use_tc_tiling_on_sc=False),
        out_type=(
            jax.ShapeDtypeStruct((S,), jnp.float32),
            jax.ShapeDtypeStruct((B,), jnp.float32),
        ),
        scratch_types=(
            pltpu.VMEM((s_per,), jnp.int32),
            pltpu.VMEM((b_per,), jnp.int32),
            pltpu.VMEM((s_per,), jnp.float32),
            pltpu.VMEM((b_per,), jnp.float32),
            pltpu.SemaphoreType.DMA,
        ),
    )
    def gk(b_hbm, sid_hbm, lab_hbm, sb_hbm, tb_hbm,
           sidx_v, lidx_v, sbias_v, lbias_v, sem):
        wid = lax.axis_index("s") * 2 + lax.axis_index("c")
        sbase = wid * s_per
        lbase = wid * b_per
        pltpu.sync_copy(sid_hbm.at[pl.ds(sbase, s_per)], sidx_v)
        pltpu.sync_copy(lab_hbm.at[pl.ds(lbase, b_per)], lidx_v)
        cps = []
        for k in range(s_per // _CH):
            sl = pl.ds(k * _CH, _CH)
            cps.append(pltpu.async_copy(
                b_hbm.at[sidx_v.at[sl]], sbias_v.at[sl], sem))
        for k in range(b_per // _CH):
            sl = pl.ds(k * _CH, _CH)
            cps.append(pltpu.async_copy(
                b_hbm.at[lidx_v.at[sl]], lbias_v.at[sl], sem))
        for cp in cps:
            cp.wait()
        pltpu.sync_copy(sbias_v, sb_hbm.at[pl.ds(sbase, s_per)])
        pltpu.sync_copy(lbias_v, tb_hbm.at[pl.ds(lbase, b_per)])

    return gk(bias, sample_ids, labels)


def _half_select(wide, parity_col):
    lo = wide[:, 0:64]
    hi = wide[:, 64:128]
    return jnp.where(parity_col == 1, hi, lo)


def _tc_body(xT_ref, sww_ref, tww_ref, sb_ref, sf_ref, sid_ref, lab_ref,
             labc_ref, tb_ref, tf_ref, out_ref, swa_scr, radd_scr, sid_scr):
    i = pl.program_id(0)

    @pl.when(i == 0)
    def _():
        swa_scr[1:, :] = _half_select(sww_ref[:], sid_ref[:] & 1)
        swa_scr[0:1, :] = jnp.zeros((1, 64), jnp.float32)
        radd_scr[1:, :] = sb_ref[:] - jnp.log(sf_ref[:])
        radd_scr[0:1, :] = jnp.zeros((1, 1), jnp.float32)
        sid_scr[1:, :] = sid_ref[:]
        sid_scr[0:1, :] = jnp.full((1, 1), -1, jnp.int32)

    xT = xT_ref[:]
    sl = lax.dot_general(swa_scr[:], xT, (((1,), (0,)), ((), ())),
                         preferred_element_type=jnp.float32)
    sl = sl + radd_scr[:]
    acc = sid_scr[:] == lab_ref[:]
    sl = jnp.where(acc, jnp.float32(-1e37), sl)
    tw = _half_select(tww_ref[:], labc_ref[:] & 1)
    twT = jnp.transpose(tw)
    tl = (jnp.sum(xT * twT, axis=0, keepdims=True) + tb_ref[:]
          - jnp.log(tf_ref[:]))
    out_ref[:] = sl
    out_ref[0:1, :] = tl


def _tc_logits_T(xT, sww, tww, sb_col, sf_col, sid_col, lab_row, lab_col,
                 tb_row, tf_row):
    D, B = xT.shape
    S = sww.shape[0]
    S1 = S + 1
    D2 = sww.shape[1]
    BN = 128
    return pl.pallas_call(
        _tc_body,
        grid=(B // BN,),
        in_specs=[
            pl.BlockSpec((D, BN), lambda i: (0, i)),
            pl.BlockSpec((S, D2), lambda i: (0, 0)),
            pl.BlockSpec((BN, D2), lambda i: (i, 0)),
            pl.BlockSpec((S, 1), lambda i: (0, 0)),
            pl.BlockSpec((S, 1), lambda i: (0, 0)),
            pl.BlockSpec((S, 1), lambda i: (0, 0)),
            pl.BlockSpec((1, BN), lambda i: (0, i)),
            pl.BlockSpec((BN, 1), lambda i: (i, 0)),
            pl.BlockSpec((1, BN), lambda i: (0, i)),
            pl.BlockSpec((1, BN), lambda i: (0, i)),
        ],
        out_specs=pl.BlockSpec((S1, BN), lambda i: (0, i)),
        out_shape=jax.ShapeDtypeStruct((S1, B), jnp.float32),
        scratch_shapes=[
            pltpu.VMEM((S1, D), jnp.float32),
            pltpu.VMEM((S1, 1), jnp.float32),
            pltpu.VMEM((S1, 1), jnp.int32),
        ],
    )(xT, sww, tww, sb_col, sf_col, sid_col, lab_row, lab_col, tb_row, tf_row)


def kernel(inputs, labels, weight, bias, sample_ids, true_freq, sample_freq):
    B = inputs.shape[0]
    V, D = weight.shape
    weight2 = weight.reshape(V // 2, 2 * D)
    sww, tww = _sc_gather_rows(weight2, sample_ids, labels)
    sb, tb = _sc_gather_bias(bias, sample_ids, labels)
    outT = _tc_logits_T(
        inputs.T, sww, tww,
        sb.reshape(-1, 1), sample_freq.reshape(-1, 1),
        sample_ids.reshape(-1, 1), labels.reshape(1, -1), labels.reshape(-1, 1),
        tb.reshape(1, -1), true_freq.reshape(1, -1))
    return (outT.T, jnp.zeros((B,), jnp.int32))


# speedup vs baseline: 2.3610x; 2.3610x over previous
import functools

import jax
import jax.numpy as jnp
from jax import lax
from jax.experimental import pallas as pl
from jax.experimental.pallas import tpu as pltpu
from jax.experimental.pallas import tpu_sc as plsc

_NW = 32
_CH = 128
_TR = 8


def _sc_gather_rows(weight3, bias, sample_ids, labels):
    VT, TR, D = weight3.shape
    S = sample_ids.shape[0]
    B = labels.shape[0]
    s_per = S // _NW
    b_per = B // _NW
    mesh = plsc.VectorSubcoreMesh(core_axis_name="c", subcore_axis_name="s")

    @functools.partial(
        pl.kernel,
        mesh=mesh,
        out_type=(
            jax.ShapeDtypeStruct((S, D), jnp.float32),
            jax.ShapeDtypeStruct((B, D), jnp.float32),
            jax.ShapeDtypeStruct((S,), jnp.float32),
            jax.ShapeDtypeStruct((B,), jnp.float32),
        ),
        scratch_types=(
            pltpu.VMEM((s_per,), jnp.int32),
            pltpu.VMEM((b_per,), jnp.int32),
            pltpu.VMEM((s_per, D), jnp.float32),
            pltpu.VMEM((b_per, D), jnp.float32),
            pltpu.VMEM((s_per,), jnp.float32),
            pltpu.VMEM((b_per,), jnp.float32),
            pltpu.SemaphoreType.DMA,
            pltpu.SemaphoreType.DMA,
            pltpu.SemaphoreType.DMA,
        ),
    )
    def gk(w_hbm, b_hbm, sid_hbm, lab_hbm, sw_hbm, tw_hbm, sb_hbm, tb_hbm,
           sidx_v, lidx_v, srows_v, lrows_v, sbias_v, lbias_v,
           sem_s, sem_l, sem_b):
        wid = lax.axis_index("s") * 2 + lax.axis_index("c")
        sbase = wid * s_per
        lbase = wid * b_per
        pltpu.sync_copy(sid_hbm.at[pl.ds(sbase, s_per)], sidx_v)
        pltpu.sync_copy(lab_hbm.at[pl.ds(lbase, b_per)], lidx_v)
        bias_cps = []
        for k in range(s_per // _CH):
            sl = pl.ds(k * _CH, _CH)
            bias_cps.append(pltpu.async_copy(
                b_hbm.at[sidx_v.at[sl]], sbias_v.at[sl], sem_b))
        for k in range(b_per // _CH):
            sl = pl.ds(k * _CH, _CH)
            bias_cps.append(pltpu.async_copy(
                b_hbm.at[lidx_v.at[sl]], lbias_v.at[sl], sem_b))

        def s_body(g, carry):
            base = g * 16
            v16 = sidx_v[pl.ds(base, 16)]
            for j in range(16):
                v = v16[j]
                pltpu.make_async_copy(
                    w_hbm.at[v >> 3, pl.ds(v & (_TR - 1), 1)],
                    srows_v.at[pl.ds(base + j, 1)], sem_s).start()
            return carry

        def l_body(g, carry):
            base = g * 16
            v16 = lidx_v[pl.ds(base, 16)]
            for j in range(16):
                v = v16[j]
                pltpu.make_async_copy(
                    w_hbm.at[v >> 3, pl.ds(v & (_TR - 1), 1)],
                    lrows_v.at[pl.ds(base + j, 1)], sem_l).start()
            return carry

        lax.fori_loop(0, s_per // 16, s_body, 0)
        lax.fori_loop(0, b_per // 16, l_body, 0)
        pltpu.make_async_copy(
            sw_hbm.at[pl.ds(sbase, s_per)], srows_v, sem_s).wait()
        pltpu.make_async_copy(
            tw_hbm.at[pl.ds(lbase, b_per)], lrows_v, sem_l).wait()
        for cp in bias_cps:
            cp.wait()
        pltpu.sync_copy(srows_v, sw_hbm.at[pl.ds(sbase, s_per)])
        pltpu.sync_copy(lrows_v, tw_hbm.at[pl.ds(lbase, b_per)])
        pltpu.sync_copy(sbias_v, sb_hbm.at[pl.ds(sbase, s_per)])
        pltpu.sync_copy(lbias_v, tb_hbm.at[pl.ds(lbase, b_per)])

    return gk(weight3, bias, sample_ids, labels)


def _sc_gather_bias(bias, sample_ids, labels):
    S = sample_ids.shape[0]
    B = labels.shape[0]
    s_per = S // _NW
    b_per = B // _NW
    mesh = plsc.VectorSubcoreMesh(core_axis_name="c", subcore_axis_name="s")

    @functools.partial(
        pl.kernel,
        mesh=mesh,
        compiler_params=pltpu.CompilerParams(use_tc_tiling_on_sc=False),
        out_type=(
            jax.ShapeDtypeStruct((S,), jnp.float32),
            jax.ShapeDtypeStruct((B,), jnp.float32),
        ),
        scratch_types=(
            pltpu.VMEM((s_per,), jnp.int32),
            pltpu.VMEM((b_per,), jnp.int32),
            pltpu.VMEM((s_per,), jnp.float32),
            pltpu.VMEM((b_per,), jnp.float32),
            pltpu.SemaphoreType.DMA,
        ),
    )
    def gk(b_hbm, sid_hbm, lab_hbm, sb_hbm, tb_hbm,
           sidx_v, lidx_v, sbias_v, lbias_v, sem):
        wid = lax.axis_index("s") * 2 + lax.axis_index("c")
        sbase = wid * s_per
        lbase = wid * b_per
        pltpu.sync_copy(sid_hbm.at[pl.ds(sbase, s_per)], sidx_v)
        pltpu.sync_copy(lab_hbm.at[pl.ds(lbase, b_per)], lidx_v)
        cps = []
        for k in range(s_per // _CH):
            sl = pl.ds(k * _CH, _CH)
            cps.append(pltpu.async_copy(
                b_hbm.at[sidx_v.at[sl]], sbias_v.at[sl], sem))
        for k in range(b_per // _CH):
            sl = pl.ds(k * _CH, _CH)
            cps.append(pltpu.async_copy(
                b_hbm.at[lidx_v.at[sl]], lbias_v.at[sl], sem))
        for cp in cps:
            cp.wait()
        pltpu.sync_copy(sbias_v, sb_hbm.at[pl.ds(sbase, s_per)])
        pltpu.sync_copy(lbias_v, tb_hbm.at[pl.ds(lbase, b_per)])

    return gk(bias, sample_ids, labels)


def _tc_body(xT_ref, sw_ref, tw_ref, sb_ref, sf_ref, sid_ref, lab_ref,
             tb_ref, tf_ref, out_ref, swa_scr, radd_scr, sid_scr):
    i = pl.program_id(0)

    @pl.when(i == 0)
    def _():
        swa_scr[1:, :] = sw_ref[:]
        swa_scr[0:1, :] = jnp.zeros((1, sw_ref.shape[1]), jnp.float32)
        radd_scr[1:, :] = sb_ref[:] - jnp.log(sf_ref[:])
        radd_scr[0:1, :] = jnp.zeros((1, 1), jnp.float32)
        sid_scr[1:, :] = sid_ref[:]
        sid_scr[0:1, :] = jnp.full((1, 1), -1, jnp.int32)

    xT = xT_ref[:]
    sl = lax.dot_general(swa_scr[:], xT, (((1,), (0,)), ((), ())),
                         preferred_element_type=jnp.float32)
    sl = sl + radd_scr[:]
    acc = sid_scr[:] == lab_ref[:]
    sl = jnp.where(acc, jnp.float32(-1e37), sl)
    twT = jnp.transpose(tw_ref[:])
    tl = (jnp.sum(xT * twT, axis=0, keepdims=True) + tb_ref[:]
          - jnp.log(tf_ref[:]))
    out_ref[:] = sl
    out_ref[0:1, :] = tl


def _tc_logits_T(xT, sw, tw, sb_col, sf_col, sid_col, lab_row, tb_row,
                 tf_row):
    D, B = xT.shape
    S = sw.shape[0]
    S1 = S + 1
    BN = 256
    return pl.pallas_call(
        _tc_body,
        grid=(B // BN,),
        in_specs=[
            pl.BlockSpec((D, BN), lambda i: (0, i)),
            pl.BlockSpec((S, D), lambda i: (0, 0)),
            pl.BlockSpec((BN, D), lambda i: (i, 0)),
            pl.BlockSpec((S, 1), lambda i: (0, 0)),
            pl.BlockSpec((S, 1), lambda i: (0, 0)),
            pl.BlockSpec((S, 1), lambda i: (0, 0)),
            pl.BlockSpec((1, BN), lambda i: (0, i)),
            pl.BlockSpec((1, BN), lambda i: (0, i)),
            pl.BlockSpec((1, BN), lambda i: (0, i)),
        ],
        out_specs=pl.BlockSpec((S1, BN), lambda i: (0, i)),
        out_shape=jax.ShapeDtypeStruct((S1, B), jnp.float32),
        scratch_shapes=[
            pltpu.VMEM((S1, D), jnp.float32),
            pltpu.VMEM((S1, 1), jnp.float32),
            pltpu.VMEM((S1, 1), jnp.int32),
        ],
    )(xT, sw, tw, sb_col, sf_col, sid_col, lab_row, tb_row, tf_row)


def kernel(inputs, labels, weight, bias, sample_ids, true_freq, sample_freq):
    B = inputs.shape[0]
    V, D = weight.shape
    weight3 = weight.reshape(V // _TR, _TR, D)
    sw, tw, sb, tb = _sc_gather_rows(weight3, bias, sample_ids, labels)
    outT = _tc_logits_T(
        inputs.T, sw, tw,
        sb.reshape(-1, 1), sample_freq.reshape(-1, 1),
        sample_ids.reshape(-1, 1), labels.reshape(1, -1),
        tb.reshape(1, -1), true_freq.reshape(1, -1))
    return (outT.T, jnp.zeros((B,), jnp.int32))


# speedup vs baseline: 2.4025x; 1.0176x over previous
import functools

import jax
import jax.numpy as jnp
from jax import lax
from jax.experimental import pallas as pl
from jax.experimental.pallas import tpu as pltpu
from jax.experimental.pallas import tpu_sc as plsc

_NW = 32
_CH = 128
_TR = 8


def _sc_gather_rows(weight3, bias, sample_ids, labels):
    VT, TR, D = weight3.shape
    S = sample_ids.shape[0]
    B = labels.shape[0]
    s_per = S // _NW
    b_per = B // _NW
    mesh = plsc.VectorSubcoreMesh(core_axis_name="c", subcore_axis_name="s")

    @functools.partial(
        pl.kernel,
        mesh=mesh,
        out_type=(
            jax.ShapeDtypeStruct((S, D), jnp.float32),
            jax.ShapeDtypeStruct((B, D), jnp.float32),
            jax.ShapeDtypeStruct((S,), jnp.float32),
            jax.ShapeDtypeStruct((B,), jnp.float32),
        ),
        scratch_types=(
            pltpu.VMEM((s_per,), jnp.int32),
            pltpu.VMEM((b_per,), jnp.int32),
            pltpu.VMEM((s_per, D), jnp.float32),
            pltpu.VMEM((b_per, D), jnp.float32),
            pltpu.VMEM((s_per,), jnp.float32),
            pltpu.VMEM((b_per,), jnp.float32),
            pltpu.SemaphoreType.DMA,
            pltpu.SemaphoreType.DMA,
            pltpu.SemaphoreType.DMA,
        ),
    )
    def gk(w_hbm, b_hbm, sid_hbm, lab_hbm, sw_hbm, tw_hbm, sb_hbm, tb_hbm,
           sidx_v, lidx_v, srows_v, lrows_v, sbias_v, lbias_v,
           sem_s, sem_l, sem_b):
        wid = lax.axis_index("s") * 2 + lax.axis_index("c")
        sbase = wid * s_per
        lbase = wid * b_per
        pltpu.sync_copy(sid_hbm.at[pl.ds(sbase, s_per)], sidx_v)
        pltpu.sync_copy(lab_hbm.at[pl.ds(lbase, b_per)], lidx_v)
        bias_cps = []
        for k in range(s_per // _CH):
            sl = pl.ds(k * _CH, _CH)
            bias_cps.append(pltpu.async_copy(
                b_hbm.at[sidx_v.at[sl]], sbias_v.at[sl], sem_b))
        for k in range(b_per // _CH):
            sl = pl.ds(k * _CH, _CH)
            bias_cps.append(pltpu.async_copy(
                b_hbm.at[lidx_v.at[sl]], lbias_v.at[sl], sem_b))

        def s_body(g, carry):
            base = g * 16
            v16 = sidx_v[pl.ds(base, 16)]
            for j in range(16):
                v = v16[j]
                pltpu.make_async_copy(
                    w_hbm.at[v >> 3, pl.ds(v & (_TR - 1), 1)],
                    srows_v.at[pl.ds(base + j, 1)], sem_s).start()
            return carry

        def l_body(g, carry):
            base = g * 16
            v16 = lidx_v[pl.ds(base, 16)]
            for j in range(16):
                v = v16[j]
                pltpu.make_async_copy(
                    w_hbm.at[v >> 3, pl.ds(v & (_TR - 1), 1)],
                    lrows_v.at[pl.ds(base + j, 1)], sem_l).start()
            return carry

        lax.fori_loop(0, s_per // 16, s_body, 0)
        lax.fori_loop(0, b_per // 16, l_body, 0)
        pltpu.make_async_copy(
            sw_hbm.at[pl.ds(sbase, s_per)], srows_v, sem_s).wait()
        pltpu.make_async_copy(
            tw_hbm.at[pl.ds(lbase, b_per)], lrows_v, sem_l).wait()
        for cp in bias_cps:
            cp.wait()
        pltpu.sync_copy(srows_v, sw_hbm.at[pl.ds(sbase, s_per)])
        pltpu.sync_copy(lrows_v, tw_hbm.at[pl.ds(lbase, b_per)])
        pltpu.sync_copy(sbias_v, sb_hbm.at[pl.ds(sbase, s_per)])
        pltpu.sync_copy(lbias_v, tb_hbm.at[pl.ds(lbase, b_per)])

    return gk(weight3, bias, sample_ids, labels)


def _sc_gather_bias(bias, sample_ids, labels):
    S = sample_ids.shape[0]
    B = labels.shape[0]
    s_per = S // _NW
    b_per = B // _NW
    mesh = plsc.VectorSubcoreMesh(core_axis_name="c", subcore_axis_name="s")

    @functools.partial(
        pl.kernel,
        mesh=mesh,
        compiler_params=pltpu.CompilerParams(use_tc_tiling_on_sc=False),
        out_type=(
            jax.ShapeDtypeStruct((S,), jnp.float32),
            jax.ShapeDtypeStruct((B,), jnp.float32),
        ),
        scratch_types=(
            pltpu.VMEM((s_per,), jnp.int32),
            pltpu.VMEM((b_per,), jnp.int32),
            pltpu.VMEM((s_per,), jnp.float32),
            pltpu.VMEM((b_per,), jnp.float32),
            pltpu.SemaphoreType.DMA,
        ),
    )
    def gk(b_hbm, sid_hbm, lab_hbm, sb_hbm, tb_hbm,
           sidx_v, lidx_v, sbias_v, lbias_v, sem):
        wid = lax.axis_index("s") * 2 + lax.axis_index("c")
        sbase = wid * s_per
        lbase = wid * b_per
        pltpu.sync_copy(sid_hbm.at[pl.ds(sbase, s_per)], sidx_v)
        pltpu.sync_copy(lab_hbm.at[pl.ds(lbase, b_per)], lidx_v)
        cps = []
        for k in range(s_per // _CH):
            sl = pl.ds(k * _CH, _CH)
            cps.append(pltpu.async_copy(
                b_hbm.at[sidx_v.at[sl]], sbias_v.at[sl], sem))
        for k in range(b_per // _CH):
            sl = pl.ds(k * _CH, _CH)
            cps.append(pltpu.async_copy(
                b_hbm.at[lidx_v.at[sl]], lbias_v.at[sl], sem))
        for cp in cps:
            cp.wait()
        pltpu.sync_copy(sbias_v, sb_hbm.at[pl.ds(sbase, s_per)])
        pltpu.sync_copy(lbias_v, tb_hbm.at[pl.ds(lbase, b_per)])

    return gk(bias, sample_ids, labels)


def _tc_body(xT_ref, sw_ref, tw_ref, sb_ref, sf_ref, sid_ref, lab_ref,
             tb_ref, tf_ref, out_ref, swa_scr, sid_scr):
    i = pl.program_id(0)
    D = sw_ref.shape[1]

    @pl.when(i == 0)
    def _():
        swa_scr[1:, 0:D] = sw_ref[:]
        swa_scr[1:, D:D + 1] = sb_ref[:] - jnp.log(sf_ref[:])
        swa_scr[0:1, :] = jnp.zeros((1, D + 1), jnp.float32)
        sid_scr[1:, :] = sid_ref[:]
        sid_scr[0:1, :] = jnp.full((1, 1), -1, jnp.int32)

    xT = xT_ref[:]
    xTa = jnp.concatenate(
        [xT, jnp.ones((1, xT.shape[1]), jnp.float32)], axis=0)
    sl = lax.dot_general(swa_scr[:], xTa, (((1,), (0,)), ((), ())),
                         preferred_element_type=jnp.float32)
    acc = sid_scr[:] == lab_ref[:]
    sl = jnp.where(acc, jnp.float32(-1e37), sl)
    twT = jnp.transpose(tw_ref[:])
    tl = (jnp.sum(xT * twT, axis=0, keepdims=True) + tb_ref[:]
          - jnp.log(tf_ref[:]))
    out_ref[:] = sl
    out_ref[0:1, :] = tl


def _tc_logits_T(xT, sw, tw, sb_col, sf_col, sid_col, lab_row, tb_row,
                 tf_row):
    D, B = xT.shape
    S = sw.shape[0]
    S1 = S + 1
    BN = 256
    return pl.pallas_call(
        _tc_body,
        grid=(B // BN,),
        in_specs=[
            pl.BlockSpec((D, BN), lambda i: (0, i)),
            pl.BlockSpec((S, D), lambda i: (0, 0)),
            pl.BlockSpec((BN, D), lambda i: (i, 0)),
            pl.BlockSpec((S, 1), lambda i: (0, 0)),
            pl.BlockSpec((S, 1), lambda i: (0, 0)),
            pl.BlockSpec((S, 1), lambda i: (0, 0)),
            pl.BlockSpec((1, BN), lambda i: (0, i)),
            pl.BlockSpec((1, BN), lambda i: (0, i)),
            pl.BlockSpec((1, BN), lambda i: (0, i)),
        ],
        out_specs=pl.BlockSpec((S1, BN), lambda i: (0, i)),
        out_shape=jax.ShapeDtypeStruct((S1, B), jnp.float32),
        scratch_shapes=[
            pltpu.VMEM((S1, D + 1), jnp.float32),
            pltpu.VMEM((S1, 1), jnp.int32),
        ],
    )(xT, sw, tw, sb_col, sf_col, sid_col, lab_row, tb_row, tf_row)


def kernel(inputs, labels, weight, bias, sample_ids, true_freq, sample_freq):
    B = inputs.shape[0]
    V, D = weight.shape
    weight3 = weight.reshape(V // _TR, _TR, D)
    sw, tw, sb, tb = _sc_gather_rows(weight3, bias, sample_ids, labels)
    outT = _tc_logits_T(
        inputs.T, sw, tw,
        sb.reshape(-1, 1), sample_freq.reshape(-1, 1),
        sample_ids.reshape(-1, 1), labels.reshape(1, -1),
        tb.reshape(1, -1), true_freq.reshape(1, -1))
    return (outT.T, jnp.zeros((B,), jnp.int32))


# speedup vs baseline: 2.4435x; 1.0170x over previous
import functools

import jax
import jax.numpy as jnp
from jax import lax
from jax.experimental import pallas as pl
from jax.experimental.pallas import tpu as pltpu
from jax.experimental.pallas import tpu_sc as plsc

_NW = 32
_CH = 128
_TR = 8


def _sc_gather_rows(weight3, bias, sample_ids, labels):
    VT, TR, D = weight3.shape
    S = sample_ids.shape[0]
    B = labels.shape[0]
    s_per = S // _NW
    b_per = B // _NW
    mesh = plsc.VectorSubcoreMesh(core_axis_name="c", subcore_axis_name="s")

    @functools.partial(
        pl.kernel,
        mesh=mesh,
        out_type=(
            jax.ShapeDtypeStruct((S, D), jnp.float32),
            jax.ShapeDtypeStruct((B, D), jnp.float32),
            jax.ShapeDtypeStruct((S,), jnp.float32),
            jax.ShapeDtypeStruct((B,), jnp.float32),
        ),
        scratch_types=(
            pltpu.VMEM((s_per,), jnp.int32),
            pltpu.VMEM((b_per,), jnp.int32),
            pltpu.VMEM((s_per, D), jnp.float32),
            pltpu.VMEM((b_per, D), jnp.float32),
            pltpu.VMEM((s_per,), jnp.float32),
            pltpu.VMEM((b_per,), jnp.float32),
            pltpu.SemaphoreType.DMA,
            pltpu.SemaphoreType.DMA,
            pltpu.SemaphoreType.DMA,
        ),
    )
    def gk(w_hbm, b_hbm, sid_hbm, lab_hbm, sw_hbm, tw_hbm, sb_hbm, tb_hbm,
           sidx_v, lidx_v, srows_v, lrows_v, sbias_v, lbias_v,
           sem_s, sem_l, sem_b):
        wid = lax.axis_index("s") * 2 + lax.axis_index("c")
        sbase = wid * s_per
        lbase = wid * b_per
        pltpu.sync_copy(sid_hbm.at[pl.ds(sbase, s_per)], sidx_v)
        pltpu.sync_copy(lab_hbm.at[pl.ds(lbase, b_per)], lidx_v)
        bias_cps = []
        for k in range(s_per // _CH):
            sl = pl.ds(k * _CH, _CH)
            bias_cps.append(pltpu.async_copy(
                b_hbm.at[sidx_v.at[sl]], sbias_v.at[sl], sem_b))
        for k in range(b_per // _CH):
            sl = pl.ds(k * _CH, _CH)
            bias_cps.append(pltpu.async_copy(
                b_hbm.at[lidx_v.at[sl]], lbias_v.at[sl], sem_b))

        def s_body(g, carry):
            base = g * 16
            v16 = sidx_v[pl.ds(base, 16)]
            for j in range(16):
                v = v16[j]
                pltpu.make_async_copy(
                    w_hbm.at[v >> 3, pl.ds(v & (_TR - 1), 1)],
                    srows_v.at[pl.ds(base + j, 1)], sem_s).start()
            return carry

        def l_body(g, carry):
            base = g * 16
            v16 = lidx_v[pl.ds(base, 16)]
            for j in range(16):
                v = v16[j]
                pltpu.make_async_copy(
                    w_hbm.at[v >> 3, pl.ds(v & (_TR - 1), 1)],
                    lrows_v.at[pl.ds(base + j, 1)], sem_l).start()
            return carry

        lax.fori_loop(0, s_per // 16, s_body, 0)
        lax.fori_loop(0, b_per // 16, l_body, 0)
        pltpu.make_async_copy(
            sw_hbm.at[pl.ds(sbase, s_per)], srows_v, sem_s).wait()
        pltpu.make_async_copy(
            tw_hbm.at[pl.ds(lbase, b_per)], lrows_v, sem_l).wait()
        for cp in bias_cps:
            cp.wait()
        pltpu.sync_copy(srows_v, sw_hbm.at[pl.ds(sbase, s_per)])
        pltpu.sync_copy(lrows_v, tw_hbm.at[pl.ds(lbase, b_per)])
        pltpu.sync_copy(sbias_v, sb_hbm.at[pl.ds(sbase, s_per)])
        pltpu.sync_copy(lbias_v, tb_hbm.at[pl.ds(lbase, b_per)])

    return gk(weight3, bias, sample_ids, labels)


def _sc_gather_bias(bias, sample_ids, labels):
    S = sample_ids.shape[0]
    B = labels.shape[0]
    s_per = S // _NW
    b_per = B // _NW
    mesh = plsc.VectorSubcoreMesh(core_axis_name="c", subcore_axis_name="s")

    @functools.partial(
        pl.kernel,
        mesh=mesh,
        compiler_params=pltpu.CompilerParams(use_tc_tiling_on_sc=False),
        out_type=(
            jax.ShapeDtypeStruct((S,), jnp.float32),
            jax.ShapeDtypeStruct((B,), jnp.float32),
        ),
        scratch_types=(
            pltpu.VMEM((s_per,), jnp.int32),
            pltpu.VMEM((b_per,), jnp.int32),
            pltpu.VMEM((s_per,), jnp.float32),
            pltpu.VMEM((b_per,), jnp.float32),
            pltpu.SemaphoreType.DMA,
        ),
    )
    def gk(b_hbm, sid_hbm, lab_hbm, sb_hbm, tb_hbm,
           sidx_v, lidx_v, sbias_v, lbias_v, sem):
        wid = lax.axis_index("s") * 2 + lax.axis_index("c")
        sbase = wid * s_per
        lbase = wid * b_per
        pltpu.sync_copy(sid_hbm.at[pl.ds(sbase, s_per)], sidx_v)
        pltpu.sync_copy(lab_hbm.at[pl.ds(lbase, b_per)], lidx_v)
        cps = []
        for k in range(s_per // _CH):
            sl = pl.ds(k * _CH, _CH)
            cps.append(pltpu.async_copy(
                b_hbm.at[sidx_v.at[sl]], sbias_v.at[sl], sem))
        for k in range(b_per // _CH):
            sl = pl.ds(k * _CH, _CH)
            cps.append(pltpu.async_copy(
                b_hbm.at[lidx_v.at[sl]], lbias_v.at[sl], sem))
        for cp in cps:
            cp.wait()
        pltpu.sync_copy(sbias_v, sb_hbm.at[pl.ds(sbase, s_per)])
        pltpu.sync_copy(lbias_v, tb_hbm.at[pl.ds(lbase, b_per)])

    return gk(bias, sample_ids, labels)


def _tc_body(xT_ref, sw_ref, tw_ref, sb_ref, sf_ref, sid_ref, lab_ref,
             tb_ref, tf_ref, out_ref, swa_scr, sid_scr):
    i = pl.program_id(0)
    D = sw_ref.shape[1]

    @pl.when(i == 0)
    def _():
        swa_scr[1:, 0:D] = sw_ref[:]
        swa_scr[1:, D:D + 1] = jnp.transpose(sb_ref[:] - jnp.log(sf_ref[:]))
        swa_scr[0:1, :] = jnp.zeros((1, D + 1), jnp.float32)
        sid_scr[1:, :] = jnp.transpose(sid_ref[:])
        sid_scr[0:1, :] = jnp.full((1, 1), -1, jnp.int32)

    xT = xT_ref[:]
    xTa = jnp.concatenate(
        [xT, jnp.ones((1, xT.shape[1]), jnp.float32)], axis=0)
    sl = lax.dot_general(swa_scr[:], xTa, (((1,), (0,)), ((), ())),
                         preferred_element_type=jnp.float32)
    acc = sid_scr[:] == lab_ref[:]
    sl = jnp.where(acc, jnp.float32(-1e37), sl)
    twT = jnp.transpose(tw_ref[:])
    tl = (jnp.sum(xT * twT, axis=0, keepdims=True) + tb_ref[:]
          - jnp.log(tf_ref[:]))
    out_ref[:] = sl
    out_ref[0:1, :] = tl


def _tc_logits_T(xT, sw, tw, sb_col, sf_col, sid_col, lab_row, tb_row,
                 tf_row):
    D, B = xT.shape
    S = sw.shape[0]
    S1 = S + 1
    BN = 256
    return pl.pallas_call(
        _tc_body,
        grid=(B // BN,),
        in_specs=[
            pl.BlockSpec((D, BN), lambda i: (0, i)),
            pl.BlockSpec((S, D), lambda i: (0, 0)),
            pl.BlockSpec((BN, D), lambda i: (i, 0)),
            pl.BlockSpec((1, S), lambda i: (0, 0)),
            pl.BlockSpec((1, S), lambda i: (0, 0)),
            pl.BlockSpec((1, S), lambda i: (0, 0)),
            pl.BlockSpec((1, BN), lambda i: (0, i)),
            pl.BlockSpec((1, BN), lambda i: (0, i)),
            pl.BlockSpec((1, BN), lambda i: (0, i)),
        ],
        out_specs=pl.BlockSpec((S1, BN), lambda i: (0, i)),
        out_shape=jax.ShapeDtypeStruct((S1, B), jnp.float32),
        scratch_shapes=[
            pltpu.VMEM((S1, D + 1), jnp.float32),
            pltpu.VMEM((S1, 1), jnp.int32),
        ],
    )(xT, sw, tw, sb_col, sf_col, sid_col, lab_row, tb_row, tf_row)


def kernel(inputs, labels, weight, bias, sample_ids, true_freq, sample_freq):
    B = inputs.shape[0]
    V, D = weight.shape
    weight3 = weight.reshape(V // _TR, _TR, D)
    sw, tw, sb, tb = _sc_gather_rows(weight3, bias, sample_ids, labels)
    outT = _tc_logits_T(
        inputs.T, sw, tw,
        sb.reshape(1, -1), sample_freq.reshape(1, -1),
        sample_ids.reshape(1, -1), labels.reshape(1, -1),
        tb.reshape(1, -1), true_freq.reshape(1, -1))
    return (outT.T, jnp.zeros((B,), jnp.int32))
